# baseline (speedup 1.0000x reference)
import functools

import jax
import jax.numpy as jnp
from jax import lax
from jax.experimental import pallas as pl
from jax.experimental.pallas import tpu as pltpu
from jax.experimental.pallas import tpu_sc as plsc

_K = 64
_D = 16
_N_ITER = 4
_BT = 8192
_NW = 32


def _assign_body(n_valid, first, x_ref, cin_ref, assign_out, cent_s):
    ib = pl.program_id(0)
    bt = x_ref.shape[1]

    @pl.when(ib == 0)
    def _():
        if first:
            cent_s[...] = cin_ref[...]
        else:
            tot = jnp.sum(cin_ref[...], axis=(0, 3))
            cnt = tot[:_K, _D:_D + 1]
            cent_s[...] = tot[:_K, :_D] / jnp.maximum(cnt, 1.0)

    xb = x_ref[...]
    c = cent_s[...]
    h = 0.5 * jnp.sum(c * c, axis=1, keepdims=True)
    scores = jax.lax.dot_general(
        c, xb, (((1,), (0,)), ((), ())), preferred_element_type=jnp.float32
    )
    dist = h - scores
    m = jnp.min(dist, axis=0, keepdims=True)
    ii = jax.lax.broadcasted_iota(jnp.int32, (_K, bt), 0)
    assign = jnp.min(jnp.where(dist == m, ii, _K), axis=0, keepdims=True)
    col = ib * bt + jax.lax.broadcasted_iota(jnp.int32, (1, bt), 1)
    assign = jnp.where(col < n_valid, assign, _K)
    assign_out[...] = assign.reshape((bt,))


def _tc_assign(xt, cin, n, npad, first):
    nb = npad // _BT
    return pl.pallas_call(
        functools.partial(_assign_body, n, first),
        grid=(nb,),
        in_specs=[
            pl.BlockSpec((_D, _BT), lambda ib: (0, ib)),
            pl.BlockSpec(cin.shape, lambda ib: (0,) * len(cin.shape)),
        ],
        out_specs=pl.BlockSpec((_BT,), lambda ib: (ib,)),
        out_shape=jax.ShapeDtypeStruct((npad,), jnp.int32),
        scratch_shapes=[pltpu.VMEM((_K, _D), jnp.float32)],
    )(xt, cin)


def _sc_body(npad, ch, xf_ref, a_ref, out_ref, xbuf, abuf, acc):
    cid = lax.axis_index("c")
    sid = lax.axis_index("s")
    wid = sid * 2 + cid
    pw = npad // _NW
    iota16 = lax.broadcasted_iota(jnp.int32, (16,), 0)
    ones16 = jnp.ones((16,), jnp.float32)
    zeros16 = jnp.zeros((16,), jnp.float32)

    def zero_step(t, carry):
        acc[pl.ds(t * 16, 16)] = zeros16
        return carry

    lax.fori_loop(0, 65 * 17, zero_step, 0)

    base_w = wid * pw
    for cidx in range(pw // ch):
        base = base_w + cidx * ch
        for j in range(16):
            pltpu.sync_copy(
                xf_ref.at[pl.ds(j * npad + base, ch)], xbuf.at[pl.ds(j * ch, ch)]
            )
        pltpu.sync_copy(a_ref.at[pl.ds(base, ch)], abuf)

        def step(i, carry):
            off = i * 16
            a272 = abuf[pl.ds(off, 16)] * 272 + iota16
            for j in range(16):
                xj = xbuf[pl.ds(j * ch + off, 16)]
                plsc.addupdate_scatter(acc, [a272 + (j * 16)], xj)
            plsc.addupdate_scatter(acc, [a272 + 256], ones16)
            return carry

        lax.fori_loop(0, ch // 16, step, 0)

    pltpu.sync_copy(acc, out_ref.at[wid])


def _sc_scatter(xf, assign, npad):
    pw = npad // _NW
    ch = pw // 8
    mesh = plsc.VectorSubcoreMesh(core_axis_name="c", subcore_axis_name="s")
    k = functools.partial(
        pl.kernel,
        mesh=mesh,
        compiler_params=pltpu.CompilerParams(needs_layout_passes=False),
        out_type=jax.ShapeDtypeStruct((_NW, 65 * 17 * 16), jnp.float32),
        scratch_types=[
            pltpu.VMEM((16 * ch,), jnp.float32),
            pltpu.VMEM((ch,), jnp.int32),
            pltpu.VMEM((65 * 17 * 16,), jnp.float32),
        ],
    )(functools.partial(_sc_body, npad, ch))
    return k(xf, assign)


def _finish_body(p_ref, cent_out, counts_out):
    tot = jnp.sum(p_ref[...], axis=(0, 3))
    cnt = tot[:_K, _D:_D + 1]
    cent_out[...] = tot[:_K, :_D] / jnp.maximum(cnt, 1.0)
    counts_out[...] = cnt


def _tc_finish(part):
    return pl.pallas_call(
        _finish_body,
        out_shape=[
            jax.ShapeDtypeStruct((_K, _D), jnp.float32),
            jax.ShapeDtypeStruct((_K, 1), jnp.float32),
        ],
    )(part)


def _mono_body(n_valid, x_ref, c0_ref, cent_out, counts_out, cent_s, sums_s):
    it = pl.program_id(0)
    ib = pl.program_id(1)
    nb = pl.num_programs(1)
    bt = x_ref.shape[1]

    @pl.when(jnp.logical_and(it == 0, ib == 0))
    def _():
        cent_s[...] = c0_ref[...]

    @pl.when(ib == 0)
    def _():
        sums_s[...] = jnp.zeros_like(sums_s)

    xb = x_ref[...]
    c = cent_s[...]
    h = 0.5 * jnp.sum(c * c, axis=1, keepdims=True)
    scores = jax.lax.dot_general(
        c, xb, (((1,), (0,)), ((), ())), preferred_element_type=jnp.float32
    )
    dist = h - scores
    m = jnp.min(dist, axis=0, keepdims=True)
    ii = jax.lax.broadcasted_iota(jnp.int32, (_K, bt), 0)
    assign = jnp.min(jnp.where(dist == m, ii, _K), axis=0, keepdims=True)
    col = ib * bt + jax.lax.broadcasted_iota(jnp.int32, (1, bt), 1)
    assign = jnp.where(col < n_valid, assign, -1)
    onehot = (ii == assign).astype(jnp.bfloat16)

    xa = jnp.concatenate([xb, jnp.ones((8, bt), jnp.float32)], axis=0)
    xh = xa.astype(jnp.bfloat16)
    xl = (xa - xh.astype(jnp.float32)).astype(jnp.bfloat16)
    sums_s[...] += jax.lax.dot_general(
        onehot, xh.T, (((1,), (0,)), ((), ())),
        preferred_element_type=jnp.float32,
    ) + jax.lax.dot_general(
        onehot, xl.T, (((1,), (0,)), ((), ())),
        preferred_element_type=jnp.float32,
    )

    @pl.when(ib == nb - 1)
    def _():
        cnt = sums_s[:, _D:_D + 1]
        newc = sums_s[:, :_D] / jnp.maximum(cnt, 1.0)
        cent_s[...] = newc

        @pl.when(it == _N_ITER - 1)
        def _():
            cent_out[...] = newc
            counts_out[...] = cnt


def _kernel_tc_monolithic(x, centroids):
    n = x.shape[0]
    nb = pl.cdiv(n, _BT)
    npad = nb * _BT - n
    xt = jnp.pad(x.T, ((0, 0), (0, npad)))
    cent, counts = pl.pallas_call(
        functools.partial(_mono_body, n),
        grid=(_N_ITER, nb),
        in_specs=[
            pl.BlockSpec((_D, _BT), lambda it, ib: (0, ib)),
            pl.BlockSpec((_K, _D), lambda it, ib: (0, 0)),
        ],
        out_specs=[
            pl.BlockSpec((_K, _D), lambda it, ib: (0, 0)),
            pl.BlockSpec((_K, 1), lambda it, ib: (0, 0)),
        ],
        out_shape=[
            jax.ShapeDtypeStruct((_K, _D), jnp.float32),
            jax.ShapeDtypeStruct((_K, 1), jnp.float32),
        ],
        scratch_shapes=[
            pltpu.VMEM((_K, _D), jnp.float32),
            pltpu.VMEM((_K, _D + 8), jnp.float32),
        ],
    )(xt, centroids)
    return cent, counts[:, 0]


def _kernel_hybrid(x, centroids):
    n = x.shape[0]
    nb = pl.cdiv(n, _BT)
    npad = nb * _BT
    xt = jnp.pad(x.T, ((0, 0), (0, npad - n)))
    xf = xt.reshape(-1)
    cin = centroids
    first = True
    part = None
    for _ in range(_N_ITER):
        assign = _tc_assign(xt, cin, n, npad, first)
        part = _sc_scatter(xf, assign, npad).reshape(_NW, 65, 17, 16)
        cin = part
        first = False
    cent, counts = _tc_finish(part)
    return cent, counts[:, 0]


def kernel(x, centroids):
    return _kernel_hybrid(x, centroids)
